# Initial kernel scaffold; baseline (speedup 1.0000x reference)
import jax
import jax.numpy as jnp
from jax.experimental import pallas as pl


def kernel(indices, table):
    raise NotImplementedError("write your pallas kernel here")


# speedup vs baseline: 13.7981x; 13.7981x over previous
import functools

import jax
import jax.numpy as jnp
from jax import lax
from jax.experimental import pallas as pl
from jax.experimental.pallas import tpu as pltpu
from jax.experimental.pallas import tpu_sc as plsc

B = 4096
L = 200
V = 100000
D = 128

NC = 2
NS = 16
LANES = 16
NW = NC * NS
BPW = B // NW
NCHUNK = 2
CH = L // NCHUNK
NBUF = 2
NVREG = D // LANES


def _bow_body(idx_hbm, table_hbm, out_hbm, idx_v, buf_v, out_v, sem0, sem1):
    wid = lax.axis_index("s") * NC + lax.axis_index("c")
    sems = (sem0, sem1)
    inv = jnp.full((LANES,), 1.0 / L, dtype=jnp.float32)

    pltpu.sync_copy(idx_hbm.at[wid], idx_v)

    def start_gather(slot, bag):
        for c in range(NCHUNK):
            pltpu.make_async_copy(
                table_hbm.at[idx_v.at[bag * NCHUNK + c]],
                buf_v.at[slot, c],
                sems[slot],
            ).start()

    def drain(slot):
        for c in range(NCHUNK):
            pltpu.make_async_copy(
                table_hbm.at[idx_v.at[0]],
                buf_v.at[slot, c],
                sems[slot],
            ).wait()

    def consume(slot, bag):
        def row_add(l, accs):
            out = []
            for k in range(NVREG):
                a = accs[k]
                for c in range(NCHUNK):
                    a = a + buf_v[slot, c, l, pl.ds(k * LANES, LANES)]
                out.append(a)
            return tuple(out)

        accs = tuple(jnp.zeros((LANES,), jnp.float32) for _ in range(NVREG))
        accs = lax.fori_loop(0, CH, row_add, accs)
        for k in range(NVREG):
            out_v[bag, pl.ds(k * LANES, LANES)] = accs[k] * inv

    for s in range(NBUF):
        start_gather(s, s)

    def step(i, _):
        for s in range(NBUF):
            bag = i * NBUF + s
            drain(s)
            consume(s, bag)
            start_gather(s, bag + NBUF)
        return 0

    lax.fori_loop(0, BPW // NBUF - 1, step, 0)

    for s in range(NBUF):
        bag = BPW - NBUF + s
        drain(s)
        consume(s, bag)

    pltpu.sync_copy(out_v, out_hbm.at[pl.ds(wid * BPW, BPW)])


@jax.jit
def _bow(idx_r, table):
    mesh = plsc.VectorSubcoreMesh(core_axis_name="c", subcore_axis_name="s")
    return pl.kernel(
        _bow_body,
        mesh=mesh,
        out_type=jax.ShapeDtypeStruct((B, D), jnp.float32),
        scratch_types=[
            pltpu.VMEM((BPW * NCHUNK, CH), jnp.int32),
            pltpu.VMEM((NBUF, NCHUNK, CH, D), jnp.float32),
            pltpu.VMEM((BPW, D), jnp.float32),
            pltpu.SemaphoreType.DMA,
            pltpu.SemaphoreType.DMA,
        ],
    )(idx_r, table)


def kernel(indices, table):
    idx_r = indices.reshape(NW, BPW * NCHUNK, CH)
    return _bow(idx_r, table)
